# baseline (speedup 1.0000x reference)
import functools

import jax
import jax.numpy as jnp
from jax import lax
from jax.experimental import pallas as pl
from jax.experimental.pallas import tpu as pltpu
from jax.experimental.pallas import tpu_sc as plsc

B, L = 4096, 200
V, D = 1000000, 64
M, T = 200, 2
N = B * L
EPS = 1e-12

NC, NS = 2, 16
NW = NC * NS
TPW = N // NW
C = 512
IDXJ = C // 128
NCHUNK = TPW // C


def _sc_kernel(x_hbm, px_hbm, tx_hbm, w_hbm, p_hbm, t_hbm, g_hbm, b_hbm,
               out_hbm, xi_v, pi_v, ti_v, wr_v, pr_v, tr_v, g_v, b_v, sem):
    wid = lax.axis_index("s") * NC + lax.axis_index("c")

    pltpu.sync_copy(g_hbm, g_v)
    pltpu.sync_copy(b_hbm, b_v)

    def tok_body(i, _):
        vs = []
        for d in range(4):
            w = wr_v[i, pl.ds(d * 16, 16)]
            p = pr_v[i, pl.ds(d * 16, 16)]
            t = tr_v[i, pl.ds(d * 16, 16)]
            vs.append(w + p + t)
        s = (vs[0] + vs[1]) + (vs[2] + vs[3])
        q = (vs[0] * vs[0] + vs[1] * vs[1]) + (vs[2] * vs[2] + vs[3] * vs[3])
        ssum = jnp.sum(s)
        qsum = jnp.sum(q)
        mu = ssum * (1.0 / 64.0)
        var = qsum * (1.0 / 64.0) - mu * mu + EPS
        bits = lax.bitcast_convert_type(var, jnp.int32)
        y = lax.bitcast_convert_type(jnp.int32(0x5F3759DF) - (bits >> 1),
                                     jnp.float32)
        for _ in range(3):
            y = y * (1.5 - 0.5 * var * y * y)
        mu_b = jnp.full((16,), mu, dtype=jnp.float32)
        rs_b = jnp.full((16,), y, dtype=jnp.float32)
        for d in range(4):
            o = (vs[d] - mu_b) * rs_b * g_v[pl.ds(d * 16, 16)] \
                + b_v[pl.ds(d * 16, 16)]
            wr_v[i, pl.ds(d * 16, 16)] = o
        return 0

    def chunk_body(c, _):
        rowb = wid * (TPW // 128) + c * IDXJ
        base = wid * TPW + c * C
        pltpu.sync_copy(x_hbm.at[pl.ds(rowb, IDXJ)], xi_v)
        pltpu.sync_copy(px_hbm.at[pl.ds(rowb, IDXJ)], pi_v)
        pltpu.sync_copy(tx_hbm.at[pl.ds(rowb, IDXJ)], ti_v)
        cps = []
        for j in range(IDXJ):
            dst = pl.ds(j * 128, 128)
            cps.append(pltpu.async_copy(w_hbm.at[xi_v.at[j]],
                                        wr_v.at[dst], sem))
            cps.append(pltpu.async_copy(p_hbm.at[pi_v.at[j]],
                                        pr_v.at[dst], sem))
            cps.append(pltpu.async_copy(t_hbm.at[ti_v.at[j]],
                                        tr_v.at[dst], sem))
        for cp in cps:
            cp.wait()
        lax.fori_loop(0, C, tok_body, 0)
        pltpu.sync_copy(wr_v, out_hbm.at[pl.ds(base, C)])
        return 0

    lax.fori_loop(0, NCHUNK, chunk_body, 0)


def kernel(x, pos_x, tok_x, word_emb, pos_emb, tok_emb, gamma, beta):
    x2 = x.reshape(N // 128, 128).astype(jnp.int32)
    p2 = pos_x.reshape(N // 128, 128).astype(jnp.int32)
    t2 = tok_x.reshape(N // 128, 128).astype(jnp.int32)

    mesh = plsc.VectorSubcoreMesh(core_axis_name="c", subcore_axis_name="s")
    run = functools.partial(
        pl.kernel,
        mesh=mesh,
        compiler_params=pltpu.CompilerParams(needs_layout_passes=False,
                                             use_tc_tiling_on_sc=False),
        out_type=jax.ShapeDtypeStruct((N, D), jnp.float32),
        scratch_types=[
            pltpu.VMEM((IDXJ, 128), jnp.int32),
            pltpu.VMEM((IDXJ, 128), jnp.int32),
            pltpu.VMEM((IDXJ, 128), jnp.int32),
            pltpu.VMEM((C, D), jnp.float32),
            pltpu.VMEM((C, D), jnp.float32),
            pltpu.VMEM((C, D), jnp.float32),
            pltpu.VMEM((D,), jnp.float32),
            pltpu.VMEM((D,), jnp.float32),
            pltpu.SemaphoreType.DMA,
        ],
    )(_sc_kernel)
    out = run(x2, p2, t2, word_emb, pos_emb, tok_emb, gamma, beta)
    return out.reshape(B, L, D)


# speedup vs baseline: 1.0007x; 1.0007x over previous
import functools

import jax
import jax.numpy as jnp
from jax import lax
from jax.experimental import pallas as pl
from jax.experimental.pallas import tpu as pltpu
from jax.experimental.pallas import tpu_sc as plsc

B, L = 4096, 200
V, D = 1000000, 64
M, T = 200, 2
N = B * L
EPS = 1e-12

NC, NS = 2, 16
NW = NC * NS
TPW = N // NW
C = 512
IDXJ = C // 128
NCHUNK = TPW // C


def _sc_kernel(x_hbm, px_hbm, tx_hbm, w_hbm, p_hbm, t_hbm, g_hbm, b_hbm,
               out_hbm, xi_v, pi_v, ti_v, wr_v, pr_v, tr_v, g_v, b_v, sem):
    wid = lax.axis_index("s") * NC + lax.axis_index("c")

    pltpu.sync_copy(g_hbm, g_v)
    pltpu.sync_copy(b_hbm, b_v)

    def tok_body(i):
        vs = []
        for d in range(4):
            w = wr_v[i, pl.ds(d * 16, 16)]
            p = pr_v[i, pl.ds(d * 16, 16)]
            t = tr_v[i, pl.ds(d * 16, 16)]
            vs.append(w + p + t)
        s = (vs[0] + vs[1]) + (vs[2] + vs[3])
        q = (vs[0] * vs[0] + vs[1] * vs[1]) + (vs[2] * vs[2] + vs[3] * vs[3])
        ssum = jnp.sum(s)
        qsum = jnp.sum(q)
        mu = ssum * (1.0 / 64.0)
        var = qsum * (1.0 / 64.0) - mu * mu + EPS
        bits = lax.bitcast_convert_type(var, jnp.int32)
        y = lax.bitcast_convert_type(jnp.int32(0x5F3759DF) - (bits >> 1),
                                     jnp.float32)
        for _ in range(3):
            y = y * (1.5 - 0.5 * var * y * y)
        mu_b = jnp.full((16,), mu, dtype=jnp.float32)
        rs_b = jnp.full((16,), y, dtype=jnp.float32)
        for d in range(4):
            o = (vs[d] - mu_b) * rs_b * g_v[pl.ds(d * 16, 16)] \
                + b_v[pl.ds(d * 16, 16)]
            wr_v[i, pl.ds(d * 16, 16)] = o

    def chunk_body(c, _):
        rowb = wid * (TPW // 128) + c * IDXJ
        base = wid * TPW + c * C
        pltpu.sync_copy(x_hbm.at[pl.ds(rowb, IDXJ)], xi_v)
        pltpu.sync_copy(px_hbm.at[pl.ds(rowb, IDXJ)], pi_v)
        pltpu.sync_copy(tx_hbm.at[pl.ds(rowb, IDXJ)], ti_v)
        cps = []
        for j in range(IDXJ):
            dst = pl.ds(j * 128, 128)
            cps.append(pltpu.async_copy(w_hbm.at[xi_v.at[j]],
                                        wr_v.at[dst], sem))
            cps.append(pltpu.async_copy(p_hbm.at[pi_v.at[j]],
                                        pr_v.at[dst], sem))
            cps.append(pltpu.async_copy(t_hbm.at[ti_v.at[j]],
                                        tr_v.at[dst], sem))
        for cp in cps:
            cp.wait()
        plsc.parallel_loop(0, C, unroll=8)(tok_body)
        pltpu.sync_copy(wr_v, out_hbm.at[pl.ds(base, C)])
        return 0

    lax.fori_loop(0, NCHUNK, chunk_body, 0)


def kernel(x, pos_x, tok_x, word_emb, pos_emb, tok_emb, gamma, beta):
    x2 = x.reshape(N // 128, 128).astype(jnp.int32)
    p2 = pos_x.reshape(N // 128, 128).astype(jnp.int32)
    t2 = tok_x.reshape(N // 128, 128).astype(jnp.int32)

    mesh = plsc.VectorSubcoreMesh(core_axis_name="c", subcore_axis_name="s")
    run = functools.partial(
        pl.kernel,
        mesh=mesh,
        compiler_params=pltpu.CompilerParams(needs_layout_passes=False,
                                             use_tc_tiling_on_sc=False),
        out_type=jax.ShapeDtypeStruct((N, D), jnp.float32),
        scratch_types=[
            pltpu.VMEM((IDXJ, 128), jnp.int32),
            pltpu.VMEM((IDXJ, 128), jnp.int32),
            pltpu.VMEM((IDXJ, 128), jnp.int32),
            pltpu.VMEM((C, D), jnp.float32),
            pltpu.VMEM((C, D), jnp.float32),
            pltpu.VMEM((C, D), jnp.float32),
            pltpu.VMEM((D,), jnp.float32),
            pltpu.VMEM((D,), jnp.float32),
            pltpu.SemaphoreType.DMA,
        ],
    )(_sc_kernel)
    out = run(x2, p2, t2, word_emb, pos_emb, tok_emb, gamma, beta)
    return out.reshape(B, L, D)


# speedup vs baseline: 6.8558x; 6.8513x over previous
import functools

import jax
import jax.numpy as jnp
from jax import lax
from jax.experimental import pallas as pl
from jax.experimental.pallas import tpu as pltpu
from jax.experimental.pallas import tpu_sc as plsc

B, L = 4096, 200
V, D = 1000000, 64
M, T = 200, 2
N = B * L
EPS = 1e-12

NC, NS = 2, 16
NW = NC * NS
TPW = N // NW
C = 512
IDXJ = C // 128
NCHUNK = TPW // C


def _sc_kernel(x_hbm, cc_hbm, w_hbm, p_hbm, t_hbm, g_hbm, b_hbm, out_hbm,
               xi0, xi1, cc0, cc1, wr0, wr1, pos_v, tok_v, pt_v, g_v, b_v,
               sem0, sem1):
    wid = lax.axis_index("s") * NC + lax.axis_index("c")

    pltpu.sync_copy(g_hbm, g_v)
    pltpu.sync_copy(b_hbm, b_v)
    pltpu.sync_copy(p_hbm, pos_v)
    pltpu.sync_copy(t_hbm, tok_v)

    @plsc.parallel_loop(0, M)
    def _build_pt(p):
        for t in range(T):
            for d in range(4):
                pt_v[pl.ds(p * (T * D) + t * D + d * 16, 16)] = (
                    pos_v[p, pl.ds(d * 16, 16)] + tok_v[t, pl.ds(d * 16, 16)])

    iota = lax.iota(jnp.int32, 16)

    def make_grp_body(cc_v, wr_v):
        def grp_body(i0, gb):
            gs, bs = gb
            cvec = cc_v[pl.ds(i0, 16)] * D
            for k in range(16):
                i = i0 + k
                base = jnp.full((16,), cvec[k], dtype=jnp.int32)
                vs = []
                for d in range(4):
                    w = wr_v[i, pl.ds(d * 16, 16)]
                    pt = plsc.load_gather(pt_v, [base + (iota + d * 16)])
                    vs.append(w + pt)
                s = (vs[0] + vs[1]) + (vs[2] + vs[3])
                q = (vs[0] * vs[0] + vs[1] * vs[1]) \
                    + (vs[2] * vs[2] + vs[3] * vs[3])
                ssum = jnp.sum(s)
                qsum = jnp.sum(q)
                mu = ssum * (1.0 / 64.0)
                var = qsum * (1.0 / 64.0) - mu * mu + EPS
                bits = lax.bitcast_convert_type(var, jnp.int32)
                y = lax.bitcast_convert_type(
                    jnp.int32(0x5F3759DF) - (bits >> 1), jnp.float32)
                for _ in range(3):
                    y = y * (1.5 - 0.5 * var * y * y)
                mu_b = jnp.full((16,), mu, dtype=jnp.float32)
                rs_b = jnp.full((16,), y, dtype=jnp.float32)
                for d in range(4):
                    o = (vs[d] - mu_b) * rs_b * gs[d] + bs[d]
                    wr_v[i, pl.ds(d * 16, 16)] = o
            return gb
        return grp_body

    def issue(c, xi_v, cc_v, wr_v, sem):
        rowb = wid * (TPW // 128) + c * IDXJ
        base = wid * TPW + c * C
        pltpu.sync_copy(x_hbm.at[pl.ds(rowb, IDXJ)], xi_v)
        pltpu.sync_copy(cc_hbm.at[pl.ds(base, C)], cc_v)
        for j in range(IDXJ):
            pltpu.async_copy(w_hbm.at[xi_v.at[j]],
                             wr_v.at[pl.ds(j * 128, 128)], sem)

    def work(c, cc_v, wr_v, sem):
        pltpu.make_async_copy(w_hbm.at[pl.ds(0, C)], wr_v, sem).wait()
        gb = (tuple(g_v[pl.ds(d * 16, 16)] for d in range(4)),
              tuple(b_v[pl.ds(d * 16, 16)] for d in range(4)))
        plsc.parallel_loop(0, C, step=16, carry=gb)(make_grp_body(cc_v, wr_v))
        pltpu.sync_copy(wr_v, out_hbm.at[pl.ds(wid * TPW + c * C, C)])

    issue(0, xi0, cc0, wr0, sem0)

    def outer(c2, _):
        c = 2 * c2
        issue(c + 1, xi1, cc1, wr1, sem1)
        work(c, cc0, wr0, sem0)

        @pl.when(c2 < NCHUNK // 2 - 1)
        def _():
            issue(c + 2, xi0, cc0, wr0, sem0)
        work(c + 1, cc1, wr1, sem1)
        return 0

    lax.fori_loop(0, NCHUNK // 2, outer, 0)


def kernel(x, pos_x, tok_x, word_emb, pos_emb, tok_emb, gamma, beta):
    x2 = x.reshape(N // 128, 128).astype(jnp.int32)
    cc = (pos_x * T + tok_x).reshape(N).astype(jnp.int32)

    mesh = plsc.VectorSubcoreMesh(core_axis_name="c", subcore_axis_name="s")
    run = functools.partial(
        pl.kernel,
        mesh=mesh,
        compiler_params=pltpu.CompilerParams(needs_layout_passes=False,
                                             use_tc_tiling_on_sc=False),
        out_type=jax.ShapeDtypeStruct((N, D), jnp.float32),
        scratch_types=[
            pltpu.VMEM((IDXJ, 128), jnp.int32),
            pltpu.VMEM((IDXJ, 128), jnp.int32),
            pltpu.VMEM((C,), jnp.int32),
            pltpu.VMEM((C,), jnp.int32),
            pltpu.VMEM((C, D), jnp.float32),
            pltpu.VMEM((C, D), jnp.float32),
            pltpu.VMEM((M, D), jnp.float32),
            pltpu.VMEM((T, D), jnp.float32),
            pltpu.VMEM((M * T * D,), jnp.float32),
            pltpu.VMEM((D,), jnp.float32),
            pltpu.VMEM((D,), jnp.float32),
            pltpu.SemaphoreType.DMA,
            pltpu.SemaphoreType.DMA,
        ],
    )(_sc_kernel)
    out = run(x2, cc, word_emb, pos_emb, tok_emb, gamma, beta)
    return out.reshape(B, L, D)


# speedup vs baseline: 10.6304x; 1.5506x over previous
import functools

import jax
import jax.numpy as jnp
from jax import lax
from jax.experimental import pallas as pl
from jax.experimental.pallas import tpu as pltpu
from jax.experimental.pallas import tpu_sc as plsc

B, L = 4096, 200
V, D = 1000000, 64
M, T = 200, 2
N = B * L
EPS = 1e-12

NC, NS = 2, 16
NW = NC * NS
TPW = N // NW
C = 640
IDXJ = C // 128
NCHUNK = TPW // C
NROW = N // 128


def _sc_kernel(x_hbm, cc_hbm, w_hbm, p_hbm, t_hbm, g_hbm, b_hbm, out_hbm,
               xi, cc, wr, pos_v, tok_v, pt_v, g_v, b_v, semi, semg, semo):
    wid = lax.axis_index("s") * NC + lax.axis_index("c")

    pltpu.sync_copy(g_hbm, g_v)
    pltpu.sync_copy(b_hbm, b_v)
    pltpu.sync_copy(p_hbm, pos_v)
    pltpu.sync_copy(t_hbm, tok_v)

    @plsc.parallel_loop(0, M)
    def _build_pt(p):
        for t in range(T):
            for d in range(4):
                pt_v[pl.ds(p * (T * D) + t * D + d * 16, 16)] = (
                    pos_v[p, pl.ds(d * 16, 16)] + tok_v[t, pl.ds(d * 16, 16)])

    iota = lax.iota(jnp.int32, 16)

    def make_tok_body(cc_v, wr3):
        def tok_body(i, gb):
            gs, bs = gb
            j = i >> 7
            r = i & 127
            c64 = cc_v[pl.ds(i, 16)][0] * D
            base = jnp.full((16,), c64, dtype=jnp.int32)
            vs = []
            for d in range(4):
                w = wr3[j, r, pl.ds(d * 16, 16)]
                pt = plsc.load_gather(pt_v, [base + (iota + d * 16)])
                vs.append(w + pt)
            s = (vs[0] + vs[1]) + (vs[2] + vs[3])
            q = (vs[0] * vs[0] + vs[1] * vs[1]) \
                + (vs[2] * vs[2] + vs[3] * vs[3])
            ssum = jnp.sum(s)
            qsum = jnp.sum(q)
            mu = ssum * (1.0 / 64.0)
            var = qsum * (1.0 / 64.0) - mu * mu + EPS
            bits = lax.bitcast_convert_type(var, jnp.int32)
            y = lax.bitcast_convert_type(
                jnp.int32(0x5F3759DF) - (bits >> 1), jnp.float32)
            for _ in range(3):
                y = y * (1.5 - 0.5 * var * y * y)
            mu_b = jnp.full((16,), mu, dtype=jnp.float32)
            rs_b = jnp.full((16,), y, dtype=jnp.float32)
            for d in range(4):
                o = (vs[d] - mu_b) * rs_b * gs[d] + bs[d]
                wr3[j, r, pl.ds(d * 16, 16)] = o
            return gb
        return tok_body

    def issue_idx(c, q):
        rowb = wid * (TPW // 128) + c * IDXJ
        base = wid * TPW + c * C
        pltpu.async_copy(x_hbm.at[pl.ds(rowb, IDXJ)], xi[q], semi[q])
        pltpu.async_copy(cc_hbm.at[pl.ds(base, C)],
                         cc[q].at[pl.ds(0, C)], semi[q])

    def drain_idx(q):
        pltpu.make_async_copy(x_hbm.at[pl.ds(0, IDXJ)], xi[q], semi[q]).wait()
        pltpu.make_async_copy(cc_hbm.at[pl.ds(0, C)],
                              cc[q].at[pl.ds(0, C)], semi[q]).wait()

    def fire_gather(q, b):
        for j in range(IDXJ):
            pltpu.async_copy(w_hbm.at[xi[q].at[j]], wr[b].at[j], semg[b])

    def drain_wr(sem, b):
        pltpu.make_async_copy(out_hbm.at[pl.ds(0, IDXJ)], wr[b], sem).wait()

    issue_idx(0, 0)
    issue_idx(1, 1)
    issue_idx(2, 2)
    drain_idx(0)
    fire_gather(0, 0)

    def outer(c4, _):
        for q in range(4):
            c = 4 * c4 + q
            b = q % 2
            drain_wr(semg[b], b)

            def _scat_done(b=b):
                drain_wr(semo[1 - b], 1 - b)
            if q == 0:
                pl.when(c4 > 0)(_scat_done)
            else:
                _scat_done()

            def _next_gather(q=q, b=b):
                drain_idx((q + 1) % 4)
                fire_gather((q + 1) % 4, 1 - b)
            if q == 3:
                pl.when(c4 < NCHUNK // 4 - 1)(_next_gather)
            else:
                _next_gather()

            gb = (tuple(g_v[pl.ds(d * 16, 16)] for d in range(4)),
                  tuple(b_v[pl.ds(d * 16, 16)] for d in range(4)))
            plsc.parallel_loop(0, C, unroll=8, carry=gb)(
                make_tok_body(cc[q], wr[b]))

            pltpu.async_copy(
                wr[b],
                out_hbm.at[pl.ds(wid * (TPW // 128) + c * IDXJ, IDXJ)],
                semo[b])

            def _issue3(c=c, q=q):
                issue_idx(c + 3, (q + 3) % 4)
            if q == 0:
                _issue3()
            else:
                pl.when(c4 < NCHUNK // 4 - 1)(_issue3)
        return 0

    lax.fori_loop(0, NCHUNK // 4, outer, 0)
    drain_wr(semo[(NCHUNK - 1) % 2], (NCHUNK - 1) % 2)


def kernel(x, pos_x, tok_x, word_emb, pos_emb, tok_emb, gamma, beta):
    x2 = x.reshape(NROW, 128).astype(jnp.int32)
    cc = (pos_x * T + tok_x).reshape(N).astype(jnp.int32)

    mesh = plsc.VectorSubcoreMesh(core_axis_name="c", subcore_axis_name="s")
    run = functools.partial(
        pl.kernel,
        mesh=mesh,
        compiler_params=pltpu.CompilerParams(needs_layout_passes=False,
                                             use_tc_tiling_on_sc=False),
        out_type=jax.ShapeDtypeStruct((NROW, 128, D), jnp.float32),
        scratch_types=[
            [pltpu.VMEM((IDXJ, 128), jnp.int32) for _ in range(4)],
            [pltpu.VMEM((C + 16,), jnp.int32) for _ in range(4)],
            [pltpu.VMEM((IDXJ, 128, D), jnp.float32) for _ in range(2)],
            pltpu.VMEM((M, D), jnp.float32),
            pltpu.VMEM((T, D), jnp.float32),
            pltpu.VMEM((M * T * D,), jnp.float32),
            pltpu.VMEM((D,), jnp.float32),
            pltpu.VMEM((D,), jnp.float32),
            [pltpu.SemaphoreType.DMA for _ in range(4)],
            [pltpu.SemaphoreType.DMA for _ in range(2)],
            [pltpu.SemaphoreType.DMA for _ in range(2)],
        ],
    )(_sc_kernel)
    out = run(x2, cc, word_emb, pos_emb, tok_emb, gamma, beta)
    return out.reshape(B, L, D)


# speedup vs baseline: 10.9672x; 1.0317x over previous
import functools

import jax
import jax.numpy as jnp
from jax import lax
from jax.experimental import pallas as pl
from jax.experimental.pallas import tpu as pltpu
from jax.experimental.pallas import tpu_sc as plsc

B, L = 4096, 200
V, D = 1000000, 64
M, T = 200, 2
N = B * L
EPS = 1e-12

NC, NS = 2, 16
NW = NC * NS
TPW = N // NW
C = 256
IDXJ = C // 128
NCHUNK = TPW // C
NROW = N // 128
NB = 4


def _sc_kernel(x_hbm, cc_hbm, w_hbm, p_hbm, t_hbm, g_hbm, b_hbm, out_hbm,
               xi, cc, wr, pos_v, tok_v, pt_v, g_v, b_v, semi, semg, semo):
    wid = lax.axis_index("s") * NC + lax.axis_index("c")

    pltpu.sync_copy(g_hbm, g_v)
    pltpu.sync_copy(b_hbm, b_v)
    pltpu.sync_copy(p_hbm, pos_v)
    pltpu.sync_copy(t_hbm, tok_v)

    @plsc.parallel_loop(0, M)
    def _build_pt(p):
        for t in range(T):
            for d in range(4):
                pt_v[pl.ds(p * (T * D) + t * D + d * 16, 16)] = (
                    pos_v[p, pl.ds(d * 16, 16)] + tok_v[t, pl.ds(d * 16, 16)])

    iota = lax.iota(jnp.int32, 16)

    def make_tok_body(cc_v, wr3):
        def tok_body(i, gb):
            gs, bs = gb
            j = i >> 7
            r = i & 127
            c64 = cc_v[pl.ds(i, 16)][0] * D
            base = jnp.full((16,), c64, dtype=jnp.int32)
            vs = []
            for d in range(4):
                w = wr3[j, r, pl.ds(d * 16, 16)]
                pt = plsc.load_gather(pt_v, [base + (iota + d * 16)])
                vs.append(w + pt)
            s = (vs[0] + vs[1]) + (vs[2] + vs[3])
            q = (vs[0] * vs[0] + vs[1] * vs[1]) \
                + (vs[2] * vs[2] + vs[3] * vs[3])
            ssum = jnp.sum(s)
            qsum = jnp.sum(q)
            mu = ssum * (1.0 / 64.0)
            var = qsum * (1.0 / 64.0) - mu * mu + EPS
            bits = lax.bitcast_convert_type(var, jnp.int32)
            y = lax.bitcast_convert_type(
                jnp.int32(0x5F3759DF) - (bits >> 1), jnp.float32)
            for _ in range(3):
                y = y * (1.5 - 0.5 * var * y * y)
            mu_b = jnp.full((16,), mu, dtype=jnp.float32)
            rs_b = jnp.full((16,), y, dtype=jnp.float32)
            for d in range(4):
                o = (vs[d] - mu_b) * rs_b * gs[d] + bs[d]
                wr3[j, r, pl.ds(d * 16, 16)] = o
            return gb
        return tok_body

    def issue_idx(c, u):
        rowb = wid * (TPW // 128) + c * IDXJ
        base = wid * TPW + c * C
        pltpu.async_copy(x_hbm.at[pl.ds(rowb, IDXJ)], xi[u], semi[u])
        pltpu.async_copy(cc_hbm.at[pl.ds(base, C)],
                         cc[u].at[pl.ds(0, C)], semi[u])

    def drain_idx(u):
        pltpu.make_async_copy(x_hbm.at[pl.ds(0, IDXJ)], xi[u], semi[u]).wait()
        pltpu.make_async_copy(cc_hbm.at[pl.ds(0, C)],
                              cc[u].at[pl.ds(0, C)], semi[u]).wait()

    def fire_word(u):
        for j in range(IDXJ):
            pltpu.async_copy(w_hbm.at[xi[u].at[j]], wr[u].at[j], semg[u])

    def drain_wr(sem, u):
        pltpu.make_async_copy(out_hbm.at[pl.ds(0, IDXJ)], wr[u], sem).wait()

    for u in range(NB):
        issue_idx(u, u)
    drain_idx(0)
    fire_word(0)
    drain_idx(1)
    fire_word(1)

    def outer(c4, _):
        for u in range(NB):
            c = NB * c4 + u
            drain_wr(semg[u], u)

            def _fire2(u=u):
                drain_wr(semo[(u + 2) % NB], (u + 2) % NB)
                drain_idx((u + 2) % NB)
                fire_word((u + 2) % NB)

            def _fire2_first(u=u):
                drain_idx((u + 2) % NB)
                fire_word((u + 2) % NB)
            if u < 2:
                pl.when(c4 > 0)(_fire2)
                pl.when(c4 == 0)(_fire2_first)
            else:
                pl.when(c4 < NCHUNK // NB - 1)(_fire2)

            gb = (tuple(g_v[pl.ds(d * 16, 16)] for d in range(4)),
                  tuple(b_v[pl.ds(d * 16, 16)] for d in range(4)))
            plsc.parallel_loop(0, C, unroll=8, carry=gb)(
                make_tok_body(cc[u], wr[u]))

            pltpu.async_copy(
                wr[u], out_hbm.at[pl.ds(wid * (TPW // 128) + c * IDXJ, IDXJ)],
                semo[u])

            pl.when(c4 < NCHUNK // NB - 1)(lambda c=c, u=u: issue_idx(
                c + NB, u))
        return 0

    lax.fori_loop(0, NCHUNK // NB, outer, 0)
    for u in range(NB):
        drain_wr(semo[u], u)


def kernel(x, pos_x, tok_x, word_emb, pos_emb, tok_emb, gamma, beta):
    x2 = x.reshape(NROW, 128).astype(jnp.int32)
    cc = (pos_x * T + tok_x).reshape(N).astype(jnp.int32)

    mesh = plsc.VectorSubcoreMesh(core_axis_name="c", subcore_axis_name="s")
    run = functools.partial(
        pl.kernel,
        mesh=mesh,
        compiler_params=pltpu.CompilerParams(needs_layout_passes=False,
                                             use_tc_tiling_on_sc=False),
        out_type=jax.ShapeDtypeStruct((NROW, 128, D), jnp.float32),
        scratch_types=[
            [pltpu.VMEM((IDXJ, 128), jnp.int32) for _ in range(NB)],
            [pltpu.VMEM((C + 16,), jnp.int32) for _ in range(NB)],
            [pltpu.VMEM((IDXJ, 128, D), jnp.float32) for _ in range(NB)],
            pltpu.VMEM((M, D), jnp.float32),
            pltpu.VMEM((T, D), jnp.float32),
            pltpu.VMEM((M * T * D,), jnp.float32),
            pltpu.VMEM((D,), jnp.float32),
            pltpu.VMEM((D,), jnp.float32),
            [pltpu.SemaphoreType.DMA for _ in range(NB)],
            [pltpu.SemaphoreType.DMA for _ in range(NB)],
            [pltpu.SemaphoreType.DMA for _ in range(NB)],
        ],
    )(_sc_kernel)
    out = run(x2, cc, word_emb, pos_emb, tok_emb, gamma, beta)
    return out.reshape(B, L, D)
